# Initial kernel scaffold; baseline (speedup 1.0000x reference)
import jax
import jax.numpy as jnp
from jax.experimental import pallas as pl


def kernel(x, edge_index, W1, b1, g1, bb1, W2, b2, g2, bb2, W3, b3):
    raise NotImplementedError("write your pallas kernel here")


# speedup vs baseline: 8.6407x; 8.6407x over previous
import functools

import jax
import jax.numpy as jnp
from jax import lax
from jax.experimental import pallas as pl
from jax.experimental.pallas import tpu as pltpu
from jax.experimental.pallas import tpu_sc as plsc

_N = 10000
_E = 160000
_NC = 2
_NS = 16
_NW = _NC * _NS
_NB = 128
_NBLK = 40
_EPW = _NB * _NBLK
_EPAD = _NW * _EPW
_NACC = 10240
_ZR = 160
_BN_EPS = 1e-5
_BR = 400
_GR = _N // _BR


def _deg_body(dst_ref, ones_ref, zeros_ref, out_ref, acc, zbuf, onesv, dstv):
    cid = lax.axis_index("c")
    sid = lax.axis_index("s")
    w = cid * _NS + sid
    pltpu.sync_copy(zeros_ref, zbuf)
    pltpu.sync_copy(ones_ref, onesv)
    for k in range(4):
        pltpu.sync_copy(zbuf, acc.at[pl.ds(sid * 640 + k * _ZR, _ZR), :])
    plsc.subcore_barrier()

    def step(i, carry):
        base = w * _EPW + i * _NB
        pltpu.sync_copy(dst_ref.at[pl.ds(base, _NB)], dstv)
        pltpu.sync_copy(onesv, acc.at[dstv], add=True)
        return carry

    lax.fori_loop(0, _NBLK, step, 0)
    plsc.subcore_barrier()
    for j in range(5):
        r0 = sid * 640 + j * _NB
        pltpu.sync_copy(acc.at[pl.ds(r0, _NB), :], onesv)
        pltpu.sync_copy(onesv, out_ref.at[pl.ds(cid * _NACC + r0, _NB), :])


def _degree_counts(dst_pad):
    ones = jnp.ones((_NB, 16), jnp.float32)
    zeros = jnp.zeros((_ZR, 16), jnp.float32)
    mesh = plsc.VectorSubcoreMesh(core_axis_name="c", subcore_axis_name="s")
    return pl.kernel(
        _deg_body,
        out_type=jax.ShapeDtypeStruct((_NC * _NACC, 16), jnp.float32),
        mesh=mesh,
        scratch_types=[
            pltpu.VMEM_SHARED((_NACC, 16), jnp.float32),
            pltpu.VMEM((_ZR, 16), jnp.float32),
            pltpu.VMEM((_NB, 16), jnp.float32),
            pltpu.VMEM((_NB,), jnp.int32),
        ],
    )(dst_pad, ones, zeros)


def _spmm_body(C, *refs):
    xs_refs = refs[:C]
    src_ref, dst_ref, zeros_ref = refs[C:C + 3]
    out_refs = refs[C + 3:2 * C + 3]
    acc, zbuf, srcv, dstv, rows, sem = refs[2 * C + 3:]
    cid = lax.axis_index("c")
    sid = lax.axis_index("s")
    w = cid * _NS + sid
    pltpu.sync_copy(zeros_ref, zbuf)
    for c in range(C):
        for k in range(4):
            pltpu.sync_copy(zbuf, acc.at[pl.ds(sid * 640 + k * _ZR, _ZR), :])
        plsc.subcore_barrier()

        def step(i, carry):
            base = w * _EPW + i * _NB
            pltpu.sync_copy(src_ref.at[pl.ds(base, _NB)], srcv)
            pltpu.sync_copy(dst_ref.at[pl.ds(base, _NB)], dstv)
            pltpu.async_copy(xs_refs[c].at[srcv], rows, sem).wait()
            pltpu.sync_copy(rows, acc.at[dstv], add=True)
            return carry

        lax.fori_loop(0, _NBLK, step, 0)
        plsc.subcore_barrier()
        for j in range(5):
            r0 = sid * 640 + j * _NB
            pltpu.sync_copy(acc.at[pl.ds(r0, _NB), :], rows)
            pltpu.sync_copy(rows, out_refs[c].at[pl.ds(cid * _NACC + r0, _NB), :])
        plsc.subcore_barrier()


def _spmm(xs_chunks, src_pad, dst_pad):
    C = len(xs_chunks)
    F = xs_chunks[0].shape[1]
    zeros = jnp.zeros((_ZR, F), jnp.float32)
    mesh = plsc.VectorSubcoreMesh(core_axis_name="c", subcore_axis_name="s")
    out = pl.kernel(
        functools.partial(_spmm_body, C),
        out_type=[jax.ShapeDtypeStruct((_NC * _NACC, F), jnp.float32)
                  for _ in range(C)],
        mesh=mesh,
        scratch_types=[
            pltpu.VMEM_SHARED((_NACC, F), jnp.float32),
            pltpu.VMEM((_ZR, F), jnp.float32),
            pltpu.VMEM((_NB,), jnp.int32),
            pltpu.VMEM((_NB,), jnp.int32),
            pltpu.VMEM((_NB, F), jnp.float32),
            pltpu.SemaphoreType.DMA,
        ],
    )(*xs_chunks, src_pad, dst_pad, zeros)
    out = out if isinstance(out, (list, tuple)) else [out]
    return [o.reshape(_NC, _NACC, F) for o in out]


def _dinv_from_deg(deg_blk):
    d = deg_blk[0, :, :1] + deg_blk[1, :, :1] + 1.0
    return lax.rsqrt(d)


_DEG_SPEC = pl.BlockSpec((_NC, _BR, 16), lambda i: (0, i, 0))


def _pre1_body(deg_ref, x_ref, xs0_ref, xs1_ref):
    dinv = _dinv_from_deg(deg_ref[...])
    xs = x_ref[...] * dinv
    xs0_ref[...] = xs[:, :128]
    xs1_ref[...] = xs[:, 128:]


def _pre1(x, deg):
    return pl.pallas_call(
        _pre1_body,
        grid=(_GR,),
        in_specs=[_DEG_SPEC, pl.BlockSpec((_BR, 256), lambda i: (i, 0))],
        out_specs=[pl.BlockSpec((_BR, 128), lambda i: (i, 0))] * 2,
        out_shape=[jax.ShapeDtypeStruct((_N, 128), jnp.float32)] * 2,
    )(deg, x)


def _mm_body(C, deg_ref, w_ref, b_ref, *refs):
    s_refs = refs[:C]
    xs_refs = refs[C:2 * C]
    h_ref, ps_ref, pq_ref = refs[2 * C:]
    dinv = _dinv_from_deg(deg_ref[...])
    parts = []
    for c in range(C):
        s = s_refs[c][0, :, :] + s_refs[c][1, :, :] + xs_refs[c][...]
        parts.append(s * dinv)
    agg = jnp.concatenate(parts, axis=1)
    z = jnp.dot(agg, w_ref[...], preferred_element_type=jnp.float32)
    h = jnp.maximum(z + b_ref[...], 0.0)
    h_ref[...] = h
    ps_ref[...] = jnp.sum(h, axis=0, keepdims=True)[None]
    pq_ref[...] = jnp.sum(h * h, axis=0, keepdims=True)[None]


def _mm(s_chunks, xs_chunks, deg, W, b):
    C = len(xs_chunks)
    F = xs_chunks[0].shape[1]
    K = C * F
    spec_s = pl.BlockSpec((_NC, _BR, F), lambda i: (0, i, 0))
    spec_xs = pl.BlockSpec((_BR, F), lambda i: (i, 0))
    return pl.pallas_call(
        functools.partial(_mm_body, C),
        grid=(_GR,),
        in_specs=(
            [_DEG_SPEC,
             pl.BlockSpec((K, 512), lambda i: (0, 0)),
             pl.BlockSpec((1, 512), lambda i: (0, 0))]
            + [spec_s] * C + [spec_xs] * C
        ),
        out_specs=[
            pl.BlockSpec((_BR, 512), lambda i: (i, 0)),
            pl.BlockSpec((1, 1, 512), lambda i: (i, 0, 0)),
            pl.BlockSpec((1, 1, 512), lambda i: (i, 0, 0)),
        ],
        out_shape=[
            jax.ShapeDtypeStruct((_N, 512), jnp.float32),
            jax.ShapeDtypeStruct((_GR, 1, 512), jnp.float32),
            jax.ShapeDtypeStruct((_GR, 1, 512), jnp.float32),
        ],
    )(deg, W, b.reshape(1, 512), *s_chunks, *xs_chunks)


def _bn_scale_body(deg_ref, ps_ref, pq_ref, g_ref, bb_ref, h_ref, *out_refs):
    dinv = _dinv_from_deg(deg_ref[...])
    mu = jnp.sum(ps_ref[...], axis=0) * (1.0 / _N)
    var = jnp.sum(pq_ref[...], axis=0) * (1.0 / _N) - mu * mu
    rstd = lax.rsqrt(var + _BN_EPS)
    hn = (h_ref[...] - mu) * (rstd * g_ref[...]) + bb_ref[...]
    xs = hn * dinv
    for c, oref in enumerate(out_refs):
        oref[...] = xs[:, c * 128:(c + 1) * 128]


def _bn_scale(h, ps, pq, g, bb, deg):
    return pl.pallas_call(
        _bn_scale_body,
        grid=(_GR,),
        in_specs=[
            _DEG_SPEC,
            pl.BlockSpec((_GR, 1, 512), lambda i: (0, 0, 0)),
            pl.BlockSpec((_GR, 1, 512), lambda i: (0, 0, 0)),
            pl.BlockSpec((1, 512), lambda i: (0, 0)),
            pl.BlockSpec((1, 512), lambda i: (0, 0)),
            pl.BlockSpec((_BR, 512), lambda i: (i, 0)),
        ],
        out_specs=[pl.BlockSpec((_BR, 128), lambda i: (i, 0))] * 4,
        out_shape=[jax.ShapeDtypeStruct((_N, 128), jnp.float32)] * 4,
    )(deg, ps, pq, g.reshape(1, 512), bb.reshape(1, 512), h)


def _bn_mm3_body(deg_ref, ps_ref, pq_ref, g_ref, bb_ref, w_ref, h_ref, out_ref):
    dinv = _dinv_from_deg(deg_ref[...])
    mu = jnp.sum(ps_ref[...], axis=0) * (1.0 / _N)
    var = jnp.sum(pq_ref[...], axis=0) * (1.0 / _N) - mu * mu
    rstd = lax.rsqrt(var + _BN_EPS)
    hn = (h_ref[...] - mu) * (rstd * g_ref[...]) + bb_ref[...]
    t = jnp.dot(hn, w_ref[...], preferred_element_type=jnp.float32)
    out_ref[...] = t * dinv


def _bn_mm3(h, ps, pq, g, bb, W3, deg):
    return pl.pallas_call(
        _bn_mm3_body,
        grid=(_GR,),
        in_specs=[
            _DEG_SPEC,
            pl.BlockSpec((_GR, 1, 512), lambda i: (0, 0, 0)),
            pl.BlockSpec((_GR, 1, 512), lambda i: (0, 0, 0)),
            pl.BlockSpec((1, 512), lambda i: (0, 0)),
            pl.BlockSpec((1, 512), lambda i: (0, 0)),
            pl.BlockSpec((512, 128), lambda i: (0, 0)),
            pl.BlockSpec((_BR, 512), lambda i: (i, 0)),
        ],
        out_specs=pl.BlockSpec((_BR, 128), lambda i: (i, 0)),
        out_shape=jax.ShapeDtypeStruct((_N, 128), jnp.float32),
    )(deg, ps, pq, g.reshape(1, 512), bb.reshape(1, 512),
      jnp.pad(W3, ((0, 0), (0, 64))), h)


def _out_body(deg_ref, s_ref, xs_ref, b_ref, o_ref):
    dinv = _dinv_from_deg(deg_ref[...])
    z = (s_ref[0, :, :64] + s_ref[1, :, :64] + xs_ref[:, :64]) * dinv + b_ref[...]
    o_ref[...] = jax.nn.sigmoid(z)


def _out(s3, xs3, deg, b3):
    return pl.pallas_call(
        _out_body,
        grid=(_GR,),
        in_specs=[
            _DEG_SPEC,
            pl.BlockSpec((_NC, _BR, 128), lambda i: (0, i, 0)),
            pl.BlockSpec((_BR, 128), lambda i: (i, 0)),
            pl.BlockSpec((1, 64), lambda i: (0, 0)),
        ],
        out_specs=pl.BlockSpec((_BR, 64), lambda i: (i, 0)),
        out_shape=jax.ShapeDtypeStruct((_N, 64), jnp.float32),
    )(deg, s3, xs3, b3.reshape(1, 64))


def kernel(x, edge_index, W1, b1, g1, bb1, W2, b2, g2, bb2, W3, b3):
    src = edge_index[0]
    dst = edge_index[1]
    pad = _EPAD - _E
    pad_src = (jnp.arange(pad, dtype=jnp.int32) * 7) % _N
    pad_dst = _N + (jnp.arange(pad, dtype=jnp.int32) % (_NACC - _N))
    src_pad = jnp.concatenate([src, pad_src])
    dst_pad = jnp.concatenate([dst, pad_dst])

    deg = _degree_counts(dst_pad).reshape(_NC, _NACC, 16)

    xs1 = _pre1(x, deg)
    s1 = _spmm(xs1, src_pad, dst_pad)
    h1, ps1, pq1 = _mm(s1, xs1, deg, W1, b1)

    xs2 = _bn_scale(h1, ps1, pq1, g1, bb1, deg)
    s2 = _spmm(xs2, src_pad, dst_pad)
    h2, ps2, pq2 = _mm(s2, xs2, deg, W2, b2)

    xs3 = _bn_mm3(h2, ps2, pq2, g2, bb2, W3, deg)
    s3 = _spmm([xs3], src_pad, dst_pad)[0]
    return _out(s3, xs3, deg, b3)


# speedup vs baseline: 13.7452x; 1.5907x over previous
import functools

import jax
import jax.numpy as jnp
from jax import lax
from jax.experimental import pallas as pl
from jax.experimental.pallas import tpu as pltpu
from jax.experimental.pallas import tpu_sc as plsc

_N = 10000
_E = 160000
_NC = 2
_NS = 16
_NW = _NC * _NS
_NB = 128
_NBLK = 40
_EPW = _NB * _NBLK
_EPAD = _NW * _EPW
_NACC = 10240
_ZR = 160
_BN_EPS = 1e-5
_BR = 400
_GR = _N // _BR


def _deg_body(dst_ref, ones_ref, zeros_ref, out_ref, acc, zbuf, onesv, dstv):
    cid = lax.axis_index("c")
    sid = lax.axis_index("s")
    w = cid * _NS + sid
    pltpu.sync_copy(zeros_ref, zbuf)
    pltpu.sync_copy(ones_ref, onesv)
    for k in range(4):
        pltpu.sync_copy(zbuf, acc.at[pl.ds(sid * 640 + k * _ZR, _ZR), :])
    plsc.subcore_barrier()

    def step(i, carry):
        base = w * _EPW + i * _NB
        pltpu.sync_copy(dst_ref.at[pl.ds(base, _NB)], dstv)
        pltpu.sync_copy(onesv, acc.at[dstv], add=True)
        return carry

    lax.fori_loop(0, _NBLK, step, 0)
    plsc.subcore_barrier()
    for j in range(5):
        r0 = sid * 640 + j * _NB
        pltpu.sync_copy(acc.at[pl.ds(r0, _NB), :], onesv)
        pltpu.sync_copy(onesv, out_ref.at[pl.ds(cid * _NACC + r0, _NB), :])


def _degree_counts(dst_pad):
    ones = jnp.ones((_NB, 16), jnp.float32)
    zeros = jnp.zeros((_ZR, 16), jnp.float32)
    mesh = plsc.VectorSubcoreMesh(core_axis_name="c", subcore_axis_name="s")
    return pl.kernel(
        _deg_body,
        out_type=jax.ShapeDtypeStruct((_NC * _NACC, 16), jnp.float32),
        mesh=mesh,
        scratch_types=[
            pltpu.VMEM_SHARED((_NACC, 16), jnp.float32),
            pltpu.VMEM((_ZR, 16), jnp.float32),
            pltpu.VMEM((_NB, 16), jnp.float32),
            pltpu.VMEM((_NB,), jnp.int32),
        ],
    )(dst_pad, ones, zeros)


_NBUF = 2


def _spmm_body(C, *refs):
    xs_refs = refs[:C]
    src_ref, dst_ref, zeros_ref = refs[C:C + 3]
    out_refs = refs[C + 3:2 * C + 3]
    acc, srcv = refs[2 * C + 3:2 * C + 5]
    dstv = refs[2 * C + 5:2 * C + 7]
    rows = refs[2 * C + 7:2 * C + 9]
    gsems = refs[2 * C + 9:2 * C + 11]
    isems = refs[2 * C + 11:2 * C + 13]
    cid = lax.axis_index("c")
    sid = lax.axis_index("s")
    w = cid * _NS + sid
    pltpu.sync_copy(src_ref.at[w], srcv)
    for c in range(C):
        pltpu.sync_copy(zeros_ref, rows[0])
        for k in range(5):
            pltpu.sync_copy(rows[0], acc.at[pl.ds(sid * 640 + k * _NB, _NB), :])
        plsc.subcore_barrier()
        gdesc = [None, None]
        idesc = [None, None]
        pltpu.sync_copy(dst_ref.at[pl.ds(w * _EPW, _NB)], dstv[0])
        gdesc[0] = pltpu.async_copy(xs_refs[c].at[srcv.at[0]], rows[0],
                                    gsems[0])
        idesc[1] = pltpu.async_copy(dst_ref.at[pl.ds(w * _EPW + _NB, _NB)],
                                    dstv[1], isems[1])
        for i in range(_NBLK):
            b = i % 2
            nb = (i + 1) % 2
            if i + 1 < _NBLK:
                idesc[nb].wait()
                gdesc[nb] = pltpu.async_copy(xs_refs[c].at[srcv.at[i + 1]],
                                             rows[nb], gsems[nb])
            gdesc[b].wait()
            pltpu.sync_copy(rows[b], acc.at[dstv[b]], add=True)
            if i + 2 < _NBLK:
                idesc[b] = pltpu.async_copy(
                    dst_ref.at[pl.ds(w * _EPW + (i + 2) * _NB, _NB)],
                    dstv[b], isems[b])
        plsc.subcore_barrier()
        for j in range(5):
            r0 = sid * 640 + j * _NB
            pltpu.sync_copy(acc.at[pl.ds(r0, _NB), :], rows[j % 2])
            pltpu.sync_copy(rows[j % 2],
                            out_refs[c].at[pl.ds(cid * _NACC + r0, _NB), :])
        plsc.subcore_barrier()


def _spmm(xs_chunks, src3, dst_pad):
    C = len(xs_chunks)
    F = xs_chunks[0].shape[1]
    zeros = jnp.zeros((_NB, F), jnp.float32)
    mesh = plsc.VectorSubcoreMesh(core_axis_name="c", subcore_axis_name="s")
    out = pl.kernel(
        functools.partial(_spmm_body, C),
        out_type=[jax.ShapeDtypeStruct((_NC * _NACC, F), jnp.float32)
                  for _ in range(C)],
        mesh=mesh,
        scratch_types=(
            [pltpu.VMEM_SHARED((_NACC, F), jnp.float32),
             pltpu.VMEM((_NBLK, _NB), jnp.int32)]
            + [pltpu.VMEM((_NB,), jnp.int32)] * 2
            + [pltpu.VMEM((_NB, F), jnp.float32)] * 2
            + [pltpu.SemaphoreType.DMA] * 4
        ),
    )(*xs_chunks, src3, dst_pad, zeros)
    out = out if isinstance(out, (list, tuple)) else [out]
    return [o.reshape(_NC, _NACC, F) for o in out]


def _dinv_from_deg(deg_blk):
    d = deg_blk[0, :, :1] + deg_blk[1, :, :1] + 1.0
    return lax.rsqrt(d)


_DEG_SPEC = pl.BlockSpec((_NC, _BR, 16), lambda i: (0, i, 0))


def _pre1_body(deg_ref, x_ref, xs0_ref, xs1_ref):
    dinv = _dinv_from_deg(deg_ref[...])
    xs = x_ref[...] * dinv
    xs0_ref[...] = xs[:, :128]
    xs1_ref[...] = xs[:, 128:]


def _pre1(x, deg):
    return pl.pallas_call(
        _pre1_body,
        grid=(_GR,),
        in_specs=[_DEG_SPEC, pl.BlockSpec((_BR, 256), lambda i: (i, 0))],
        out_specs=[pl.BlockSpec((_BR, 128), lambda i: (i, 0))] * 2,
        out_shape=[jax.ShapeDtypeStruct((_N, 128), jnp.float32)] * 2,
    )(deg, x)


def _mm_body(C, deg_ref, w_ref, b_ref, *refs):
    s_refs = refs[:C]
    xs_refs = refs[C:2 * C]
    h_ref, ps_ref, pq_ref = refs[2 * C:]
    dinv = _dinv_from_deg(deg_ref[...])
    parts = []
    for c in range(C):
        s = s_refs[c][0, :, :] + s_refs[c][1, :, :] + xs_refs[c][...]
        parts.append(s * dinv)
    agg = jnp.concatenate(parts, axis=1)
    z = jnp.dot(agg, w_ref[...], preferred_element_type=jnp.float32)
    h = jnp.maximum(z + b_ref[...], 0.0)
    h_ref[...] = h
    ps_ref[...] = jnp.sum(h, axis=0, keepdims=True)[None]
    pq_ref[...] = jnp.sum(h * h, axis=0, keepdims=True)[None]


def _mm(s_chunks, xs_chunks, deg, W, b):
    C = len(xs_chunks)
    F = xs_chunks[0].shape[1]
    K = C * F
    spec_s = pl.BlockSpec((_NC, _BR, F), lambda i: (0, i, 0))
    spec_xs = pl.BlockSpec((_BR, F), lambda i: (i, 0))
    return pl.pallas_call(
        functools.partial(_mm_body, C),
        grid=(_GR,),
        in_specs=(
            [_DEG_SPEC,
             pl.BlockSpec((K, 512), lambda i: (0, 0)),
             pl.BlockSpec((1, 512), lambda i: (0, 0))]
            + [spec_s] * C + [spec_xs] * C
        ),
        out_specs=[
            pl.BlockSpec((_BR, 512), lambda i: (i, 0)),
            pl.BlockSpec((1, 1, 512), lambda i: (i, 0, 0)),
            pl.BlockSpec((1, 1, 512), lambda i: (i, 0, 0)),
        ],
        out_shape=[
            jax.ShapeDtypeStruct((_N, 512), jnp.float32),
            jax.ShapeDtypeStruct((_GR, 1, 512), jnp.float32),
            jax.ShapeDtypeStruct((_GR, 1, 512), jnp.float32),
        ],
    )(deg, W, b.reshape(1, 512), *s_chunks, *xs_chunks)


def _bn_scale_body(deg_ref, ps_ref, pq_ref, g_ref, bb_ref, h_ref, *out_refs):
    dinv = _dinv_from_deg(deg_ref[...])
    mu = jnp.sum(ps_ref[...], axis=0) * (1.0 / _N)
    var = jnp.sum(pq_ref[...], axis=0) * (1.0 / _N) - mu * mu
    rstd = lax.rsqrt(var + _BN_EPS)
    hn = (h_ref[...] - mu) * (rstd * g_ref[...]) + bb_ref[...]
    xs = hn * dinv
    for c, oref in enumerate(out_refs):
        oref[...] = xs[:, c * 128:(c + 1) * 128]


def _bn_scale(h, ps, pq, g, bb, deg):
    return pl.pallas_call(
        _bn_scale_body,
        grid=(_GR,),
        in_specs=[
            _DEG_SPEC,
            pl.BlockSpec((_GR, 1, 512), lambda i: (0, 0, 0)),
            pl.BlockSpec((_GR, 1, 512), lambda i: (0, 0, 0)),
            pl.BlockSpec((1, 512), lambda i: (0, 0)),
            pl.BlockSpec((1, 512), lambda i: (0, 0)),
            pl.BlockSpec((_BR, 512), lambda i: (i, 0)),
        ],
        out_specs=[pl.BlockSpec((_BR, 128), lambda i: (i, 0))] * 4,
        out_shape=[jax.ShapeDtypeStruct((_N, 128), jnp.float32)] * 4,
    )(deg, ps, pq, g.reshape(1, 512), bb.reshape(1, 512), h)


def _bn_mm3_body(deg_ref, ps_ref, pq_ref, g_ref, bb_ref, w_ref, h_ref, out_ref):
    dinv = _dinv_from_deg(deg_ref[...])
    mu = jnp.sum(ps_ref[...], axis=0) * (1.0 / _N)
    var = jnp.sum(pq_ref[...], axis=0) * (1.0 / _N) - mu * mu
    rstd = lax.rsqrt(var + _BN_EPS)
    hn = (h_ref[...] - mu) * (rstd * g_ref[...]) + bb_ref[...]
    t = jnp.dot(hn, w_ref[...], preferred_element_type=jnp.float32)
    out_ref[...] = t * dinv


def _bn_mm3(h, ps, pq, g, bb, W3, deg):
    return pl.pallas_call(
        _bn_mm3_body,
        grid=(_GR,),
        in_specs=[
            _DEG_SPEC,
            pl.BlockSpec((_GR, 1, 512), lambda i: (0, 0, 0)),
            pl.BlockSpec((_GR, 1, 512), lambda i: (0, 0, 0)),
            pl.BlockSpec((1, 512), lambda i: (0, 0)),
            pl.BlockSpec((1, 512), lambda i: (0, 0)),
            pl.BlockSpec((512, 128), lambda i: (0, 0)),
            pl.BlockSpec((_BR, 512), lambda i: (i, 0)),
        ],
        out_specs=pl.BlockSpec((_BR, 128), lambda i: (i, 0)),
        out_shape=jax.ShapeDtypeStruct((_N, 128), jnp.float32),
    )(deg, ps, pq, g.reshape(1, 512), bb.reshape(1, 512),
      jnp.pad(W3, ((0, 0), (0, 64))), h)


def _out_body(deg_ref, s_ref, xs_ref, b_ref, o_ref):
    dinv = _dinv_from_deg(deg_ref[...])
    z = (s_ref[0, :, :64] + s_ref[1, :, :64] + xs_ref[:, :64]) * dinv + b_ref[...]
    o_ref[...] = jax.nn.sigmoid(z)


def _out(s3, xs3, deg, b3):
    return pl.pallas_call(
        _out_body,
        grid=(_GR,),
        in_specs=[
            _DEG_SPEC,
            pl.BlockSpec((_NC, _BR, 128), lambda i: (0, i, 0)),
            pl.BlockSpec((_BR, 128), lambda i: (i, 0)),
            pl.BlockSpec((1, 64), lambda i: (0, 0)),
        ],
        out_specs=pl.BlockSpec((_BR, 64), lambda i: (i, 0)),
        out_shape=jax.ShapeDtypeStruct((_N, 64), jnp.float32),
    )(deg, s3, xs3, b3.reshape(1, 64))


def kernel(x, edge_index, W1, b1, g1, bb1, W2, b2, g2, bb2, W3, b3):
    src = edge_index[0]
    dst = edge_index[1]
    pad = _EPAD - _E
    pad_src = (jnp.arange(pad, dtype=jnp.int32) * 7) % _N
    pad_dst = _N + (jnp.arange(pad, dtype=jnp.int32) % (_NACC - _N))
    src3 = jnp.concatenate([src, pad_src]).reshape(_NW, _NBLK, _NB)
    dst_pad = jnp.concatenate([dst, pad_dst])

    deg = _degree_counts(dst_pad).reshape(_NC, _NACC, 16)

    xs1 = _pre1(x, deg)
    s1 = _spmm(xs1, src3, dst_pad)
    h1, ps1, pq1 = _mm(s1, xs1, deg, W1, b1)

    xs2 = _bn_scale(h1, ps1, pq1, g1, bb1, deg)
    s2 = _spmm(xs2, src3, dst_pad)
    h2, ps2, pq2 = _mm(s2, xs2, deg, W2, b2)

    xs3 = _bn_mm3(h2, ps2, pq2, g2, bb2, W3, deg)
    s3 = _spmm([xs3], src3, dst_pad)[0]
    return _out(s3, xs3, deg, b3)


# speedup vs baseline: 13.9265x; 1.0132x over previous
import functools

import jax
import jax.numpy as jnp
from jax import lax
from jax.experimental import pallas as pl
from jax.experimental.pallas import tpu as pltpu
from jax.experimental.pallas import tpu_sc as plsc

_N = 10000
_E = 160000
_NC = 2
_NS = 16
_NW = _NC * _NS
_NB = 128
_NBLK = 40
_EPW = _NB * _NBLK
_EPAD = _NW * _EPW
_NACC = 10240
_ZR = 160
_BN_EPS = 1e-5
_BR = 400
_GR = _N // _BR


def _deg_body(dst_ref, ones_ref, zeros_ref, out_ref, acc, zbuf, onesv, dstv):
    cid = lax.axis_index("c")
    sid = lax.axis_index("s")
    w = cid * _NS + sid
    pltpu.sync_copy(zeros_ref, zbuf)
    pltpu.sync_copy(ones_ref, onesv)
    for k in range(4):
        pltpu.sync_copy(zbuf, acc.at[pl.ds(sid * 640 + k * _ZR, _ZR), :])
    plsc.subcore_barrier()

    def step(i, carry):
        base = w * _EPW + i * _NB
        pltpu.sync_copy(dst_ref.at[pl.ds(base, _NB)], dstv)
        pltpu.sync_copy(onesv, acc.at[dstv], add=True)
        return carry

    lax.fori_loop(0, _NBLK, step, 0)
    plsc.subcore_barrier()
    for j in range(5):
        r0 = sid * 640 + j * _NB
        pltpu.sync_copy(acc.at[pl.ds(r0, _NB), :], onesv)
        pltpu.sync_copy(onesv, out_ref.at[pl.ds(cid * _NACC + r0, _NB), :])


def _degree_counts(dst_pad):
    ones = jnp.ones((_NB, 16), jnp.float32)
    zeros = jnp.zeros((_ZR, 16), jnp.float32)
    mesh = plsc.VectorSubcoreMesh(core_axis_name="c", subcore_axis_name="s")
    return pl.kernel(
        _deg_body,
        out_type=jax.ShapeDtypeStruct((_NC * _NACC, 16), jnp.float32),
        mesh=mesh,
        scratch_types=[
            pltpu.VMEM_SHARED((_NACC, 16), jnp.float32),
            pltpu.VMEM((_ZR, 16), jnp.float32),
            pltpu.VMEM((_NB, 16), jnp.float32),
            pltpu.VMEM((_NB,), jnp.int32),
        ],
    )(dst_pad, ones, zeros)


_NBUF = 2


def _spmm_body(C, *refs):
    xs_refs = refs[:C]
    src_ref, dst_ref, zeros_ref = refs[C:C + 3]
    out_refs = refs[C + 3:2 * C + 3]
    acc, srcv = refs[2 * C + 3:2 * C + 5]
    dstv = refs[2 * C + 5:2 * C + 9]
    rows = refs[2 * C + 9:2 * C + 11]
    gsems = refs[2 * C + 11:2 * C + 13]
    isems = refs[2 * C + 13:2 * C + 15]
    ssems = refs[2 * C + 15:2 * C + 17]
    cid = lax.axis_index("c")
    sid = lax.axis_index("s")
    w = cid * _NS + sid
    pltpu.sync_copy(src_ref.at[w], srcv)
    pltpu.sync_copy(zeros_ref, rows[0])
    for k in range(5):
        pltpu.sync_copy(rows[0], acc.at[pl.ds(sid * 640 + k * _NB, _NB), :])
    for c in range(C):
        plsc.subcore_barrier()
        gdesc = [None, None]
        sdesc = [None, None]
        idesc = [None] * 4
        pltpu.sync_copy(dst_ref.at[pl.ds(w * _EPW, _NB)], dstv[0])
        gdesc[0] = pltpu.async_copy(xs_refs[c].at[srcv.at[0]], rows[0],
                                    gsems[0])
        idesc[1] = pltpu.async_copy(dst_ref.at[pl.ds(w * _EPW + _NB, _NB)],
                                    dstv[1], isems[1])
        for i in range(_NBLK):
            b = i % 2
            nb = (i + 1) % 2
            if i + 1 < _NBLK:
                idesc[(i + 1) % 4].wait()
                if i >= 1:
                    sdesc[nb].wait()
                gdesc[nb] = pltpu.async_copy(xs_refs[c].at[srcv.at[i + 1]],
                                             rows[nb], gsems[nb])
            gdesc[b].wait()
            sdesc[b] = pltpu.async_copy(rows[b], acc.at[dstv[i % 4]],
                                        ssems[b], add=True)
            if i + 2 < _NBLK:
                idesc[(i + 2) % 4] = pltpu.async_copy(
                    dst_ref.at[pl.ds(w * _EPW + (i + 2) * _NB, _NB)],
                    dstv[(i + 2) % 4], isems[(i + 2) % 2])
        sdesc[(_NBLK - 2) % 2].wait()
        sdesc[(_NBLK - 1) % 2].wait()
        plsc.subcore_barrier()
        for j in range(5):
            r0 = sid * 640 + j * _NB
            pltpu.sync_copy(acc.at[pl.ds(r0, _NB), :], rows[j % 2])
            pltpu.sync_copy(rows[j % 2],
                            out_refs[c].at[pl.ds(cid * _NACC + r0, _NB), :])
            if c + 1 < C:
                pltpu.sync_copy(zeros_ref, acc.at[pl.ds(r0, _NB), :])


def _spmm(xs_chunks, src3, dst_pad):
    C = len(xs_chunks)
    F = xs_chunks[0].shape[1]
    zeros = jnp.zeros((_NB, F), jnp.float32)
    mesh = plsc.VectorSubcoreMesh(core_axis_name="c", subcore_axis_name="s")
    out = pl.kernel(
        functools.partial(_spmm_body, C),
        out_type=[jax.ShapeDtypeStruct((_NC * _NACC, F), jnp.float32)
                  for _ in range(C)],
        mesh=mesh,
        scratch_types=(
            [pltpu.VMEM_SHARED((_NACC, F), jnp.float32),
             pltpu.VMEM((_NBLK, _NB), jnp.int32)]
            + [pltpu.VMEM((_NB,), jnp.int32)] * 4
            + [pltpu.VMEM((_NB, F), jnp.float32)] * 2
            + [pltpu.SemaphoreType.DMA] * 6
        ),
    )(*xs_chunks, src3, dst_pad, zeros)
    out = out if isinstance(out, (list, tuple)) else [out]
    return [o.reshape(_NC, _NACC, F) for o in out]


def _dinv_from_deg(deg_blk):
    d = deg_blk[0, :, :1] + deg_blk[1, :, :1] + 1.0
    return lax.rsqrt(d)


_DEG_SPEC = pl.BlockSpec((_NC, _BR, 16), lambda i: (0, i, 0))


def _pre1_body(deg_ref, x_ref, xs0_ref, xs1_ref):
    dinv = _dinv_from_deg(deg_ref[...])
    xs = x_ref[...] * dinv
    xs0_ref[...] = xs[:, :128]
    xs1_ref[...] = xs[:, 128:]


def _pre1(x, deg):
    return pl.pallas_call(
        _pre1_body,
        grid=(_GR,),
        in_specs=[_DEG_SPEC, pl.BlockSpec((_BR, 256), lambda i: (i, 0))],
        out_specs=[pl.BlockSpec((_BR, 128), lambda i: (i, 0))] * 2,
        out_shape=[jax.ShapeDtypeStruct((_N, 128), jnp.float32)] * 2,
    )(deg, x)


def _mm_body(C, deg_ref, w_ref, b_ref, *refs):
    s_refs = refs[:C]
    xs_refs = refs[C:2 * C]
    h_ref, ps_ref, pq_ref = refs[2 * C:]
    dinv = _dinv_from_deg(deg_ref[...])
    parts = []
    for c in range(C):
        s = s_refs[c][0, :, :] + s_refs[c][1, :, :] + xs_refs[c][...]
        parts.append(s * dinv)
    agg = jnp.concatenate(parts, axis=1)
    z = jnp.dot(agg, w_ref[...], preferred_element_type=jnp.float32)
    h = jnp.maximum(z + b_ref[...], 0.0)
    h_ref[...] = h
    ps_ref[...] = jnp.sum(h, axis=0, keepdims=True)[None]
    pq_ref[...] = jnp.sum(h * h, axis=0, keepdims=True)[None]


def _mm(s_chunks, xs_chunks, deg, W, b):
    C = len(xs_chunks)
    F = xs_chunks[0].shape[1]
    K = C * F
    spec_s = pl.BlockSpec((_NC, _BR, F), lambda i: (0, i, 0))
    spec_xs = pl.BlockSpec((_BR, F), lambda i: (i, 0))
    return pl.pallas_call(
        functools.partial(_mm_body, C),
        grid=(_GR,),
        in_specs=(
            [_DEG_SPEC,
             pl.BlockSpec((K, 512), lambda i: (0, 0)),
             pl.BlockSpec((1, 512), lambda i: (0, 0))]
            + [spec_s] * C + [spec_xs] * C
        ),
        out_specs=[
            pl.BlockSpec((_BR, 512), lambda i: (i, 0)),
            pl.BlockSpec((1, 1, 512), lambda i: (i, 0, 0)),
            pl.BlockSpec((1, 1, 512), lambda i: (i, 0, 0)),
        ],
        out_shape=[
            jax.ShapeDtypeStruct((_N, 512), jnp.float32),
            jax.ShapeDtypeStruct((_GR, 1, 512), jnp.float32),
            jax.ShapeDtypeStruct((_GR, 1, 512), jnp.float32),
        ],
    )(deg, W, b.reshape(1, 512), *s_chunks, *xs_chunks)


def _bn_scale_body(deg_ref, ps_ref, pq_ref, g_ref, bb_ref, h_ref, *out_refs):
    dinv = _dinv_from_deg(deg_ref[...])
    mu = jnp.sum(ps_ref[...], axis=0) * (1.0 / _N)
    var = jnp.sum(pq_ref[...], axis=0) * (1.0 / _N) - mu * mu
    rstd = lax.rsqrt(var + _BN_EPS)
    hn = (h_ref[...] - mu) * (rstd * g_ref[...]) + bb_ref[...]
    xs = hn * dinv
    for c, oref in enumerate(out_refs):
        oref[...] = xs[:, c * 128:(c + 1) * 128]


def _bn_scale(h, ps, pq, g, bb, deg):
    return pl.pallas_call(
        _bn_scale_body,
        grid=(_GR,),
        in_specs=[
            _DEG_SPEC,
            pl.BlockSpec((_GR, 1, 512), lambda i: (0, 0, 0)),
            pl.BlockSpec((_GR, 1, 512), lambda i: (0, 0, 0)),
            pl.BlockSpec((1, 512), lambda i: (0, 0)),
            pl.BlockSpec((1, 512), lambda i: (0, 0)),
            pl.BlockSpec((_BR, 512), lambda i: (i, 0)),
        ],
        out_specs=[pl.BlockSpec((_BR, 128), lambda i: (i, 0))] * 4,
        out_shape=[jax.ShapeDtypeStruct((_N, 128), jnp.float32)] * 4,
    )(deg, ps, pq, g.reshape(1, 512), bb.reshape(1, 512), h)


def _bn_mm3_body(deg_ref, ps_ref, pq_ref, g_ref, bb_ref, w_ref, h_ref, out_ref):
    dinv = _dinv_from_deg(deg_ref[...])
    mu = jnp.sum(ps_ref[...], axis=0) * (1.0 / _N)
    var = jnp.sum(pq_ref[...], axis=0) * (1.0 / _N) - mu * mu
    rstd = lax.rsqrt(var + _BN_EPS)
    hn = (h_ref[...] - mu) * (rstd * g_ref[...]) + bb_ref[...]
    t = jnp.dot(hn, w_ref[...], preferred_element_type=jnp.float32)
    out_ref[...] = t * dinv


def _bn_mm3(h, ps, pq, g, bb, W3, deg):
    return pl.pallas_call(
        _bn_mm3_body,
        grid=(_GR,),
        in_specs=[
            _DEG_SPEC,
            pl.BlockSpec((_GR, 1, 512), lambda i: (0, 0, 0)),
            pl.BlockSpec((_GR, 1, 512), lambda i: (0, 0, 0)),
            pl.BlockSpec((1, 512), lambda i: (0, 0)),
            pl.BlockSpec((1, 512), lambda i: (0, 0)),
            pl.BlockSpec((512, 128), lambda i: (0, 0)),
            pl.BlockSpec((_BR, 512), lambda i: (i, 0)),
        ],
        out_specs=pl.BlockSpec((_BR, 128), lambda i: (i, 0)),
        out_shape=jax.ShapeDtypeStruct((_N, 128), jnp.float32),
    )(deg, ps, pq, g.reshape(1, 512), bb.reshape(1, 512),
      jnp.pad(W3, ((0, 0), (0, 64))), h)


def _out_body(deg_ref, s_ref, xs_ref, b_ref, o_ref):
    dinv = _dinv_from_deg(deg_ref[...])
    z = (s_ref[0, :, :64] + s_ref[1, :, :64] + xs_ref[:, :64]) * dinv + b_ref[...]
    o_ref[...] = jax.nn.sigmoid(z)


def _out(s3, xs3, deg, b3):
    return pl.pallas_call(
        _out_body,
        grid=(_GR,),
        in_specs=[
            _DEG_SPEC,
            pl.BlockSpec((_NC, _BR, 128), lambda i: (0, i, 0)),
            pl.BlockSpec((_BR, 128), lambda i: (i, 0)),
            pl.BlockSpec((1, 64), lambda i: (0, 0)),
        ],
        out_specs=pl.BlockSpec((_BR, 64), lambda i: (i, 0)),
        out_shape=jax.ShapeDtypeStruct((_N, 64), jnp.float32),
    )(deg, s3, xs3, b3.reshape(1, 64))


def kernel(x, edge_index, W1, b1, g1, bb1, W2, b2, g2, bb2, W3, b3):
    src = edge_index[0]
    dst = edge_index[1]
    pad = _EPAD - _E
    pad_src = (jnp.arange(pad, dtype=jnp.int32) * 7) % _N
    pad_dst = _N + (jnp.arange(pad, dtype=jnp.int32) % (_NACC - _N))
    src3 = jnp.concatenate([src, pad_src]).reshape(_NW, _NBLK, _NB)
    dst_pad = jnp.concatenate([dst, pad_dst])

    deg = _degree_counts(dst_pad).reshape(_NC, _NACC, 16)

    xs1 = _pre1(x, deg)
    s1 = _spmm(xs1, src3, dst_pad)
    h1, ps1, pq1 = _mm(s1, xs1, deg, W1, b1)

    xs2 = _bn_scale(h1, ps1, pq1, g1, bb1, deg)
    s2 = _spmm(xs2, src3, dst_pad)
    h2, ps2, pq2 = _mm(s2, xs2, deg, W2, b2)

    xs3 = _bn_mm3(h2, ps2, pq2, g2, bb2, W3, deg)
    s3 = _spmm([xs3], src3, dst_pad)[0]
    return _out(s3, xs3, deg, b3)
